# baseline (speedup 1.0000x reference)
import functools

import jax
import jax.numpy as jnp
from jax import lax
from jax.experimental import pallas as pl
from jax.experimental.pallas import tpu as pltpu
from jax.experimental.pallas import tpu_sc as plsc

N, E, D = 10000, 320000, 128
G, GF = 64, 16
H, P = 256, 128

NC, NS = 2, 16
CHUNK = 125
DC = 64
N_CHUNKS = E // NS // CHUNK
ROWS_A = 624
ROWS_LAST = N - (NS - 1) * ROWS_A

_MESH = plsc.VectorSubcoreMesh(core_axis_name="c", subcore_axis_name="s")

NBUF = 5


def _edge_loop(table_hbm, src_v, dst_v, bufs, agg_sh, gsems, ssems):
    n_groups = N_CHUNKS // NBUF

    for k in range(NBUF):
        pltpu.async_copy(table_hbm.at[src_v.at[k]], bufs[k], gsems[k])

    def group(i, carry):
        base = i * NBUF
        for k in range(NBUF):
            j = base + k
            pltpu.make_async_copy(table_hbm.at[src_v.at[j]], bufs[k],
                                  gsems[k]).wait()
            pltpu.async_copy(bufs[k], agg_sh.at[dst_v.at[j]], ssems[k],
                             add=True)

        @pl.when(i < n_groups - 1)
        def _():
            for k in range(NBUF):
                j = base + k
                pltpu.make_async_copy(bufs[k], agg_sh.at[dst_v.at[j]],
                                      ssems[k]).wait()
                pltpu.async_copy(table_hbm.at[src_v.at[j + NBUF]], bufs[k],
                                 gsems[k])
        return carry

    lax.fori_loop(0, n_groups, group, 0)
    for k in range(NBUF):
        j = (n_groups - 1) * NBUF + k
        pltpu.make_async_copy(bufs[k], agg_sh.at[dst_v.at[j]], ssems[k]).wait()


def _zero_slice(zeros_hbm, agg_sh, s):
    @pl.when(s < NS - 1)
    def _():
        pltpu.sync_copy(zeros_hbm.at[pl.ds(0, ROWS_A)],
                        agg_sh.at[pl.ds(s * ROWS_A, ROWS_A)])

    @pl.when(s == NS - 1)
    def _():
        pltpu.sync_copy(zeros_hbm, agg_sh.at[pl.ds((NS - 1) * ROWS_A, ROWS_LAST)])


def _writeback_col(agg_sh, out_view, col, s):
    @pl.when(s < NS - 1)
    def _():
        pltpu.sync_copy(agg_sh.at[pl.ds(s * ROWS_A, ROWS_A)],
                        out_view.at[pl.ds(s * ROWS_A, ROWS_A), col])

    @pl.when(s == NS - 1)
    def _():
        pltpu.sync_copy(agg_sh.at[pl.ds((NS - 1) * ROWS_A, ROWS_LAST)],
                        out_view.at[pl.ds((NS - 1) * ROWS_A, ROWS_LAST), col])


@functools.partial(
    pl.kernel,
    out_type=jax.ShapeDtypeStruct((N, NC, DC), jnp.float32),
    mesh=_MESH,
    compiler_params=pltpu.CompilerParams(use_tc_tiling_on_sc=False),
    scratch_types=[
        pltpu.VMEM((N_CHUNKS, CHUNK), jnp.int32),
        pltpu.VMEM((N_CHUNKS, CHUNK), jnp.int32),
        [pltpu.VMEM((CHUNK, DC), jnp.float32)] * NBUF,
        pltpu.VMEM_SHARED((N, DC), jnp.float32),
        [pltpu.SemaphoreType.DMA] * NBUF,
        [pltpu.SemaphoreType.DMA] * NBUF,
    ],
)
def _sc_agg(t_hbm, src_hbm, dst_hbm, zeros_hbm, out_hbm,
            src_v, dst_v, bufs, agg_sh, gsems, ssems):
    c = lax.axis_index("c")
    s = lax.axis_index("s")
    _zero_slice(zeros_hbm, agg_sh, s)
    pltpu.sync_copy(src_hbm.at[s], src_v)
    pltpu.sync_copy(dst_hbm.at[s], dst_v)
    plsc.subcore_barrier()
    view = t_hbm.at[pl.ds(c, NC * N - (NC - 1))]
    _edge_loop(view, src_v, dst_v, bufs, agg_sh, gsems, ssems)
    plsc.subcore_barrier()
    _writeback_col(agg_sh, out_hbm, c, s)


@functools.partial(
    pl.kernel,
    out_type=jax.ShapeDtypeStruct((2, N, NC, DC), jnp.float32),
    mesh=_MESH,
    compiler_params=pltpu.CompilerParams(use_tc_tiling_on_sc=False),
    scratch_types=[
        pltpu.VMEM((N_CHUNKS, CHUNK), jnp.int32),
        pltpu.VMEM((N_CHUNKS, CHUNK), jnp.int32),
        [pltpu.VMEM((CHUNK, DC), jnp.float32)] * NBUF,
        pltpu.VMEM_SHARED((N, DC), jnp.float32),
        [pltpu.SemaphoreType.DMA] * NBUF,
        [pltpu.SemaphoreType.DMA] * NBUF,
    ],
)
def _sc_agg2(ha_hbm, hb_hbm, src_hbm, dst_hbm, zeros_hbm,
             out_hbm, src_v, dst_v, bufs, agg_sh, gsems, ssems):
    c = lax.axis_index("c")
    s = lax.axis_index("s")
    _zero_slice(zeros_hbm, agg_sh, s)
    pltpu.sync_copy(src_hbm.at[s], src_v)
    pltpu.sync_copy(dst_hbm.at[s], dst_v)
    plsc.subcore_barrier()
    nrows = NC * N - (NC - 1)
    _edge_loop(ha_hbm.at[pl.ds(c, nrows)], src_v, dst_v, bufs, agg_sh,
               gsems, ssems)
    plsc.subcore_barrier()
    _writeback_col(agg_sh, out_hbm.at[0], c, s)
    _zero_slice(zeros_hbm, agg_sh, s)
    plsc.subcore_barrier()
    _edge_loop(hb_hbm.at[pl.ds(c, nrows)], src_v, dst_v, bufs, agg_sh,
               gsems, ssems)
    plsc.subcore_barrier()
    _writeback_col(agg_sh, out_hbm.at[1], c, s)


BLK = 1000
NB = N // BLK

NEG = -3.4e38


def _layer_norm(h, g, b):
    m = jnp.mean(h, axis=-1, keepdims=True)
    v = jnp.mean((h - m) * (h - m), axis=-1, keepdims=True)
    return (h - m) * lax.rsqrt(v + 1e-5) * g + b


def _mlp0_body(x_ref, p_ref, w1_ref, b1_ref, w2_ref, b2_ref, g_ref, be_ref,
               ha_ref, hb_ref):
    hin = x_ref[...] + p_ref[...]
    a = jnp.maximum(
        jnp.dot(hin, w1_ref[...], preferred_element_type=jnp.float32) + b1_ref[...],
        0.0)
    h = jnp.dot(a, w2_ref[...], preferred_element_type=jnp.float32) + b2_ref[...]
    h = jnp.maximum(_layer_norm(h, g_ref[...], be_ref[...]), 0.0)
    ha_ref[...] = h[:, :128]
    hb_ref[...] = h[:, 128:]


def _tc_mlp0(x, agg0c, w1, b1, w2, b2, g, be):
    full = lambda shape: pl.BlockSpec(shape, lambda j: (0,) * len(shape))
    half = pl.BlockSpec((BLK, 128), lambda j: (j, 0))
    return pl.pallas_call(
        _mlp0_body,
        grid=(NB,),
        in_specs=[
            pl.BlockSpec((BLK, D), lambda j: (j, 0)),
            pl.BlockSpec((BLK, D), lambda j: (j, 0)),
            full((D, H)), full((1, H)), full((H, H)), full((1, H)),
            full((1, H)), full((1, H)),
        ],
        out_specs=[half, half],
        out_shape=[jax.ShapeDtypeStruct((N, 128), jnp.float32)] * 2,
    )(x, agg0c, w1, b1, w2, b2, g, be)


def _mlp1pool_body(ha_ref, hb_ref, p_ref, b_ref, gf_ref,
                   w3_ref, b3_ref, w4_ref, b4_ref, g_ref, be_ref,
                   wg1_ref, bg1_ref, wg2_ref, bg2_ref,
                   wc1_ref, bc1_ref, wc2_ref, bc2_ref,
                   out_ref, h2_s, gate_s, m_s, s_s, pooled_s):
    ph = pl.program_id(0)
    j = pl.program_id(1)
    b = b_ref[0, 0, :]
    iota = lax.broadcasted_iota(jnp.int32, (1, G), 1).astype(jnp.float32)
    onehot = (b[:, None] == iota).astype(jnp.float32)

    @pl.when((ph == 0) & (j == 0))
    def _():
        m_s[...] = jnp.full((1, G), NEG, jnp.float32)
        s_s[...] = jnp.zeros((1, G), jnp.float32)

    @pl.when(ph == 0)
    def _():
        hin_l = ha_ref[...] + p_ref[0]
        hin_r = hb_ref[...] + p_ref[1]
        a = jnp.maximum(
            jnp.dot(hin_l, w3_ref[:128, :], preferred_element_type=jnp.float32)
            + jnp.dot(hin_r, w3_ref[128:, :], preferred_element_type=jnp.float32)
            + b3_ref[...], 0.0)
        h2 = jnp.dot(a, w4_ref[...], preferred_element_type=jnp.float32)
        h2 = h2 + b4_ref[...]
        h2 = jnp.maximum(_layer_norm(h2, g_ref[...], be_ref[...]), 0.0)
        h2_s[j] = h2
        ag = jnp.maximum(
            jnp.dot(h2, wg1_ref[...], preferred_element_type=jnp.float32)
            + bg1_ref[...], 0.0)
        gate = (jnp.dot(ag, wg2_ref[...], preferred_element_type=jnp.float32)
                + bg2_ref[...])
        gate_s[j] = gate
        bm = jnp.max(jnp.where(onehot > 0, gate, NEG), axis=0, keepdims=True)
        m_old = m_s[...]
        m_new = jnp.maximum(m_old, bm)
        gmax_node = jnp.sum(onehot * m_new, axis=1, keepdims=True)
        e = jnp.exp(gate - gmax_node)
        s_s[...] = (s_s[...] * jnp.exp(m_old - m_new)
                    + jnp.sum(onehot * e, axis=0, keepdims=True))
        m_s[...] = m_new

    @pl.when((ph == 1) & (j == 0))
    def _():
        pooled_s[...] = jnp.zeros((G, H), jnp.float32)

    @pl.when(ph == 1)
    def _():
        gate = gate_s[j]
        gmax_node = jnp.sum(onehot * m_s[...], axis=1, keepdims=True)
        e = jnp.exp(gate - gmax_node)
        s_node = jnp.sum(onehot * s_s[...], axis=1, keepdims=True)
        alpha = e / (s_node + 1e-16)
        pooled_s[...] += lax.dot_general(
            onehot, alpha * h2_s[j], (((0,), (0,)), ((), ())),
            preferred_element_type=jnp.float32)

    @pl.when((ph == 1) & (j == NB - 1))
    def _():
        z = (jnp.dot(pooled_s[...], wc1_ref[:H, :],
                     preferred_element_type=jnp.float32)
             + jnp.dot(gf_ref[...], wc1_ref[H:, :],
                       preferred_element_type=jnp.float32)
             + bc1_ref[...])
        z = jnp.maximum(z, 0.0)
        out_ref[...] = (
            jnp.dot(z, wc2_ref[...], preferred_element_type=jnp.float32)
            + bc2_ref[...])


def _tc_mlp1pool(ha, hb, agg1r, batch3, gf, w3, b3, w4, b4, g, be,
                 wg1, bg1, wg2, bg2, wc1, bc1, wc2, bc2):
    full = lambda shape: pl.BlockSpec(shape, lambda ph, j: (0,) * len(shape))
    half = pl.BlockSpec((BLK, 128), lambda ph, j: (j * (1 - ph), 0))
    agg_spec = pl.BlockSpec((2, BLK, 128), lambda ph, j: (0, j * (1 - ph), 0))
    return pl.pallas_call(
        _mlp1pool_body,
        grid=(2, NB),
        in_specs=[
            half, half, agg_spec,
            pl.BlockSpec((1, 1, BLK), lambda ph, j: (j, 0, 0)),
            full((G, GF)),
            full((H, H)), full((1, H)), full((H, H)), full((1, H)),
            full((1, H)), full((1, H)),
            full((H, P)), full((1, P)), full((P, 1)), full((1, 1)),
            full((H + GF, P)), full((1, P)), full((P, 2)), full((1, 2)),
        ],
        out_specs=pl.BlockSpec((G, 2), lambda ph, j: (0, 0)),
        out_shape=jax.ShapeDtypeStruct((G, 2), jnp.float32),
        scratch_shapes=[
            pltpu.VMEM((NB, BLK, H), jnp.float32),
            pltpu.VMEM((NB, BLK, 1), jnp.float32),
            pltpu.VMEM((1, G), jnp.float32),
            pltpu.VMEM((1, G), jnp.float32),
            pltpu.VMEM((G, H), jnp.float32),
        ],
    )(ha, hb, agg1r, batch3, gf, w3, b3, w4, b4, g, be,
      wg1, bg1, wg2, bg2, wc1, bc1, wc2, bc2)


def kernel(x, edge_index, edge_attr, batch, global_features,
           W1, b1, W2, b2, W3, b3, W4, b4,
           ln0_g, ln0_b, ln1_g, ln1_b,
           Wg1, bg1, Wg2, bg2, Wc1, bc1, Wc2, bc2):
    src, dst = edge_index[0], edge_index[1]
    srcL0 = (src * 2).reshape(NS, N_CHUNKS, CHUNK)
    dst3 = dst.reshape(NS, N_CHUNKS, CHUNK)
    zeros = jnp.zeros((ROWS_LAST, DC), jnp.float32)

    x2 = x.reshape(NC * N, DC)
    agg0c = _sc_agg(x2, srcL0, dst3, zeros).reshape(N, D)
    ha, hb = _tc_mlp0(x, agg0c, W1, b1.reshape(1, H), W2,
                      b2.reshape(1, H), ln0_g.reshape(1, H),
                      ln0_b.reshape(1, H))
    agg1r = _sc_agg2(ha.reshape(NC * N, DC), hb.reshape(NC * N, DC),
                     srcL0, dst3, zeros).reshape(2, N, 128)
    batch3 = batch.astype(jnp.float32).reshape(NB, 1, BLK)
    return _tc_mlp1pool(ha, hb, agg1r, batch3, global_features,
                        W3, b3.reshape(1, H), W4, b4.reshape(1, H),
                        ln1_g.reshape(1, H), ln1_b.reshape(1, H),
                        Wg1, bg1.reshape(1, P), Wg2, bg2.reshape(1, 1),
                        Wc1, bc1.reshape(1, P), Wc2, bc2.reshape(1, 2))


# speedup vs baseline: 1.1334x; 1.1334x over previous
import functools

import jax
import jax.numpy as jnp
from jax import lax
from jax.experimental import pallas as pl
from jax.experimental.pallas import tpu as pltpu
from jax.experimental.pallas import tpu_sc as plsc

N, E, D = 10000, 320000, 128
G, GF = 64, 16
H, P = 256, 128

NC, NS = 2, 16
CHUNK = 125
DC = 64
N_CHUNKS = E // NS // CHUNK
ROWS_A = 624
ROWS_LAST = N - (NS - 1) * ROWS_A

_MESH = plsc.VectorSubcoreMesh(core_axis_name="c", subcore_axis_name="s")

NBUF = 5


def _edge_loop(table_hbm, src_v, dst_v, bufs, agg_sh, gsems, ssems):
    n_groups = N_CHUNKS // NBUF

    for k in range(NBUF):
        pltpu.async_copy(table_hbm.at[src_v.at[k]], bufs[k], gsems[k])

    def group(i, carry):
        base = i * NBUF
        for k in range(NBUF):
            j = base + k
            pltpu.make_async_copy(table_hbm.at[src_v.at[j]], bufs[k],
                                  gsems[k]).wait()
            pltpu.async_copy(bufs[k], agg_sh.at[dst_v.at[j]], ssems[k],
                             add=True)

        @pl.when(i < n_groups - 1)
        def _():
            for k in range(NBUF):
                j = base + k
                pltpu.make_async_copy(bufs[k], agg_sh.at[dst_v.at[j]],
                                      ssems[k]).wait()
                pltpu.async_copy(table_hbm.at[src_v.at[j + NBUF]], bufs[k],
                                 gsems[k])
        return carry

    lax.fori_loop(0, n_groups, group, 0)
    for k in range(NBUF):
        j = (n_groups - 1) * NBUF + k
        pltpu.make_async_copy(bufs[k], agg_sh.at[dst_v.at[j]], ssems[k]).wait()


def _zero_slice(zeros_hbm, agg_sh, s):
    @pl.when(s < NS - 1)
    def _():
        pltpu.sync_copy(zeros_hbm.at[pl.ds(0, ROWS_A)],
                        agg_sh.at[pl.ds(s * ROWS_A, ROWS_A)])

    @pl.when(s == NS - 1)
    def _():
        pltpu.sync_copy(zeros_hbm, agg_sh.at[pl.ds((NS - 1) * ROWS_A, ROWS_LAST)])


def _writeback_slice(agg_sh, out_hbm, c, s):
    @pl.when(s < NS - 1)
    def _():
        pltpu.sync_copy(agg_sh.at[pl.ds(s * ROWS_A, ROWS_A)],
                        out_hbm.at[c, pl.ds(s * ROWS_A, ROWS_A)])

    @pl.when(s == NS - 1)
    def _():
        pltpu.sync_copy(agg_sh.at[pl.ds((NS - 1) * ROWS_A, ROWS_LAST)],
                        out_hbm.at[c, pl.ds((NS - 1) * ROWS_A, ROWS_LAST)])


@functools.partial(
    pl.kernel,
    out_type=jax.ShapeDtypeStruct((NC, N, DC), jnp.float32),
    mesh=_MESH,
    compiler_params=pltpu.CompilerParams(use_tc_tiling_on_sc=False),
    scratch_types=[
        pltpu.VMEM((N_CHUNKS, CHUNK), jnp.int32),
        pltpu.VMEM((N_CHUNKS, CHUNK), jnp.int32),
        [pltpu.VMEM((CHUNK, DC), jnp.float32)] * NBUF,
        pltpu.VMEM_SHARED((N, DC), jnp.float32),
        [pltpu.SemaphoreType.DMA] * NBUF,
        [pltpu.SemaphoreType.DMA] * NBUF,
    ],
)
def _sc_agg(t_hbm, src_hbm, dst_hbm, zeros_hbm, out_hbm,
            src_v, dst_v, bufs, agg_sh, gsems, ssems):
    c = lax.axis_index("c")
    s = lax.axis_index("s")
    _zero_slice(zeros_hbm, agg_sh, s)
    pltpu.sync_copy(src_hbm.at[s], src_v)
    pltpu.sync_copy(dst_hbm.at[s], dst_v)
    plsc.subcore_barrier()
    view = t_hbm.at[pl.ds(c, NC * N - (NC - 1))]
    _edge_loop(view, src_v, dst_v, bufs, agg_sh, gsems, ssems)
    plsc.subcore_barrier()
    _writeback_slice(agg_sh, out_hbm, c, s)


@functools.partial(
    pl.kernel,
    out_type=jax.ShapeDtypeStruct((4, N, DC), jnp.float32),
    mesh=_MESH,
    compiler_params=pltpu.CompilerParams(use_tc_tiling_on_sc=False),
    scratch_types=[
        pltpu.VMEM((N_CHUNKS, CHUNK), jnp.int32),
        pltpu.VMEM((N_CHUNKS, CHUNK), jnp.int32),
        [pltpu.VMEM((CHUNK, DC), jnp.float32)] * NBUF,
        pltpu.VMEM_SHARED((N, DC), jnp.float32),
        [pltpu.SemaphoreType.DMA] * NBUF,
        [pltpu.SemaphoreType.DMA] * NBUF,
    ],
)
def _sc_agg2(t0_hbm, t1_hbm, t2_hbm, t3_hbm, src_hbm, dst_hbm, zeros_hbm,
             out_hbm, src_v, dst_v, bufs, agg_sh, gsems, ssems):
    c = lax.axis_index("c")
    s = lax.axis_index("s")
    _zero_slice(zeros_hbm, agg_sh, s)
    pltpu.sync_copy(src_hbm.at[s], src_v)
    pltpu.sync_copy(dst_hbm.at[s], dst_v)
    plsc.subcore_barrier()

    @pl.when(c == 0)
    def _():
        _edge_loop(t0_hbm, src_v, dst_v, bufs, agg_sh, gsems, ssems)

    @pl.when(c == 1)
    def _():
        _edge_loop(t1_hbm, src_v, dst_v, bufs, agg_sh, gsems, ssems)

    plsc.subcore_barrier()
    _writeback_slice(agg_sh, out_hbm, c, s)
    _zero_slice(zeros_hbm, agg_sh, s)
    plsc.subcore_barrier()

    @pl.when(c == 0)
    def _():
        _edge_loop(t2_hbm, src_v, dst_v, bufs, agg_sh, gsems, ssems)

    @pl.when(c == 1)
    def _():
        _edge_loop(t3_hbm, src_v, dst_v, bufs, agg_sh, gsems, ssems)

    plsc.subcore_barrier()
    _writeback_slice(agg_sh, out_hbm, c + 2, s)


BLK = 1000
NB = N // BLK

NEG = -3.4e38


def _layer_norm(h, g, b):
    m = jnp.mean(h, axis=-1, keepdims=True)
    v = jnp.mean((h - m) * (h - m), axis=-1, keepdims=True)
    return (h - m) * lax.rsqrt(v + 1e-5) * g + b


def _mlp0_body(x_ref, p_ref, w1_ref, b1_ref, w2_ref, b2_ref, g_ref, be_ref,
               h0_ref, h1_ref, h2_ref, h3_ref):
    hin = x_ref[...] + jnp.concatenate([p_ref[0], p_ref[1]], axis=1)
    a = jnp.maximum(
        jnp.dot(hin, w1_ref[...], preferred_element_type=jnp.float32) + b1_ref[...],
        0.0)
    h = jnp.dot(a, w2_ref[...], preferred_element_type=jnp.float32) + b2_ref[...]
    h = jnp.maximum(_layer_norm(h, g_ref[...], be_ref[...]), 0.0)
    h0_ref[...] = h[:, 0 * DC:1 * DC]
    h1_ref[...] = h[:, 1 * DC:2 * DC]
    h2_ref[...] = h[:, 2 * DC:3 * DC]
    h3_ref[...] = h[:, 3 * DC:4 * DC]


def _tc_mlp0(x, agg0, w1, b1, w2, b2, g, be):
    full = lambda shape: pl.BlockSpec(shape, lambda j: (0,) * len(shape))
    slab = pl.BlockSpec((BLK, DC), lambda j: (j, 0))
    return pl.pallas_call(
        _mlp0_body,
        grid=(NB,),
        in_specs=[
            pl.BlockSpec((BLK, D), lambda j: (j, 0)),
            pl.BlockSpec((NC, BLK, DC), lambda j: (0, j, 0)),
            full((D, H)), full((1, H)), full((H, H)), full((1, H)),
            full((1, H)), full((1, H)),
        ],
        out_specs=[slab, slab, slab, slab],
        out_shape=[jax.ShapeDtypeStruct((N, DC), jnp.float32)] * 4,
    )(x, agg0, w1, b1, w2, b2, g, be)


def _mlp1pool_body(h0_ref, h1_ref, h2_ref, h3_ref, p_ref, b_ref, gf_ref,
                   w3_ref, b3_ref, w4_ref, b4_ref, g_ref, be_ref,
                   wg1_ref, bg1_ref, wg2_ref, bg2_ref,
                   wc1_ref, bc1_ref, wc2_ref, bc2_ref,
                   out_ref, h2_s, gate_s, m_s, s_s, pooled_s):
    ph = pl.program_id(0)
    j = pl.program_id(1)
    b = b_ref[0, 0, :]
    iota = lax.broadcasted_iota(jnp.int32, (1, G), 1).astype(jnp.float32)
    onehot = (b[:, None] == iota).astype(jnp.float32)

    @pl.when((ph == 0) & (j == 0))
    def _():
        m_s[...] = jnp.full((1, G), NEG, jnp.float32)
        s_s[...] = jnp.zeros((1, G), jnp.float32)

    @pl.when(ph == 0)
    def _():
        hin = jnp.concatenate(
            [h0_ref[...] + p_ref[0], h1_ref[...] + p_ref[1],
             h2_ref[...] + p_ref[2], h3_ref[...] + p_ref[3]], axis=1)
        a = jnp.maximum(
            jnp.dot(hin, w3_ref[...], preferred_element_type=jnp.float32)
            + b3_ref[...], 0.0)
        h2 = jnp.dot(a, w4_ref[...], preferred_element_type=jnp.float32)
        h2 = h2 + b4_ref[...]
        h2 = jnp.maximum(_layer_norm(h2, g_ref[...], be_ref[...]), 0.0)
        h2_s[j] = h2
        ag = jnp.maximum(
            jnp.dot(h2, wg1_ref[...], preferred_element_type=jnp.float32)
            + bg1_ref[...], 0.0)
        gate = (jnp.dot(ag, wg2_ref[...], preferred_element_type=jnp.float32)
                + bg2_ref[...])
        gate_s[j] = gate
        bm = jnp.max(jnp.where(onehot > 0, gate, NEG), axis=0, keepdims=True)
        m_old = m_s[...]
        m_new = jnp.maximum(m_old, bm)
        gmax_node = jnp.sum(onehot * m_new, axis=1, keepdims=True)
        e = jnp.exp(gate - gmax_node)
        s_s[...] = (s_s[...] * jnp.exp(m_old - m_new)
                    + jnp.sum(onehot * e, axis=0, keepdims=True))
        m_s[...] = m_new

    @pl.when((ph == 1) & (j == 0))
    def _():
        pooled_s[...] = jnp.zeros((G, H), jnp.float32)

    @pl.when(ph == 1)
    def _():
        gate = gate_s[j]
        gmax_node = jnp.sum(onehot * m_s[...], axis=1, keepdims=True)
        e = jnp.exp(gate - gmax_node)
        s_node = jnp.sum(onehot * s_s[...], axis=1, keepdims=True)
        alpha = e / (s_node + 1e-16)
        pooled_s[...] += lax.dot_general(
            onehot, alpha * h2_s[j], (((0,), (0,)), ((), ())),
            preferred_element_type=jnp.float32)

    @pl.when((ph == 1) & (j == NB - 1))
    def _():
        z = (jnp.dot(pooled_s[...], wc1_ref[:H, :],
                     preferred_element_type=jnp.float32)
             + jnp.dot(gf_ref[...], wc1_ref[H:, :],
                       preferred_element_type=jnp.float32)
             + bc1_ref[...])
        z = jnp.maximum(z, 0.0)
        out_ref[...] = (
            jnp.dot(z, wc2_ref[...], preferred_element_type=jnp.float32)
            + bc2_ref[...])


def _tc_mlp1pool(h0, h1, h2, h3, agg1, batch3, gf, w3, b3, w4, b4, g, be,
                 wg1, bg1, wg2, bg2, wc1, bc1, wc2, bc2):
    full = lambda shape: pl.BlockSpec(shape, lambda ph, j: (0,) * len(shape))
    slab = pl.BlockSpec((BLK, DC), lambda ph, j: (j * (1 - ph), 0))
    agg_spec = pl.BlockSpec((4, BLK, DC), lambda ph, j: (0, j * (1 - ph), 0))
    return pl.pallas_call(
        _mlp1pool_body,
        grid=(2, NB),
        in_specs=[
            slab, slab, slab, slab, agg_spec,
            pl.BlockSpec((1, 1, BLK), lambda ph, j: (j, 0, 0)),
            full((G, GF)),
            full((H, H)), full((1, H)), full((H, H)), full((1, H)),
            full((1, H)), full((1, H)),
            full((H, P)), full((1, P)), full((P, 1)), full((1, 1)),
            full((H + GF, P)), full((1, P)), full((P, 2)), full((1, 2)),
        ],
        out_specs=pl.BlockSpec((G, 2), lambda ph, j: (0, 0)),
        out_shape=jax.ShapeDtypeStruct((G, 2), jnp.float32),
        scratch_shapes=[
            pltpu.VMEM((NB, BLK, H), jnp.float32),
            pltpu.VMEM((NB, BLK, 1), jnp.float32),
            pltpu.VMEM((1, G), jnp.float32),
            pltpu.VMEM((1, G), jnp.float32),
            pltpu.VMEM((G, H), jnp.float32),
        ],
    )(h0, h1, h2, h3, agg1, batch3, gf, w3, b3, w4, b4, g, be,
      wg1, bg1, wg2, bg2, wc1, bc1, wc2, bc2)


def kernel(x, edge_index, edge_attr, batch, global_features,
           W1, b1, W2, b2, W3, b3, W4, b4,
           ln0_g, ln0_b, ln1_g, ln1_b,
           Wg1, bg1, Wg2, bg2, Wc1, bc1, Wc2, bc2):
    src, dst = edge_index[0], edge_index[1]
    srcL0 = (src * 2).reshape(NS, N_CHUNKS, CHUNK)
    src3 = src.reshape(NS, N_CHUNKS, CHUNK)
    dst3 = dst.reshape(NS, N_CHUNKS, CHUNK)
    zeros = jnp.zeros((ROWS_LAST, DC), jnp.float32)

    x2 = x.reshape(NC * N, DC)
    agg0 = _sc_agg(x2, srcL0, dst3, zeros)
    h0, h1, h2s, h3 = _tc_mlp0(x, agg0, W1, b1.reshape(1, H), W2,
                               b2.reshape(1, H), ln0_g.reshape(1, H),
                               ln0_b.reshape(1, H))
    agg1 = _sc_agg2(h0, h1, h2s, h3, src3, dst3, zeros)
    batch3 = batch.astype(jnp.float32).reshape(NB, 1, BLK)
    return _tc_mlp1pool(h0, h1, h2s, h3, agg1, batch3, global_features,
                        W3, b3.reshape(1, H), W4, b4.reshape(1, H),
                        ln1_g.reshape(1, H), ln1_b.reshape(1, H),
                        Wg1, bg1.reshape(1, P), Wg2, bg2.reshape(1, 1),
                        Wc1, bc1.reshape(1, P), Wc2, bc2.reshape(1, 2))
